# baseline (speedup 1.0000x reference)
import functools

import jax
import jax.numpy as jnp
from jax import lax
from jax.experimental import pallas as pl
from jax.experimental.pallas import tpu as pltpu
from jax.experimental.pallas import tpu_sc as plsc

N = 10000
E = 320000
D = 128
DE = 16
G = 512
T = 10

NC = 2
NS = 16
NW = NC * NS
DH = D // 2
CHT = 128
NCHT = 158
EPAD = NS * NCHT * CHT
NP = 10240
RPT = NP // NS
RCH = 128
LANES = 16


_EB = EPAD // 32


def _encode_body(attr_ref, we_ref, out_ref):
    res = jnp.dot(attr_ref[...], we_ref[...],
                  preferred_element_type=jnp.float32)
    out_ref[0] = res[:, :DH]
    out_ref[1] = res[:, DH:]


def _encode(edge_attr, W_e):
    return pl.pallas_call(
        _encode_body,
        grid=(EPAD // _EB,),
        in_specs=[
            pl.BlockSpec((_EB, DE), lambda i: (i, 0)),
            pl.BlockSpec((DE, D), lambda i: (0, 0)),
        ],
        out_specs=pl.BlockSpec((2, _EB, DH), lambda i: (0, i, 0)),
        out_shape=jax.ShapeDtypeStruct((2, EPAD, DH), jnp.float32),
    )(edge_attr, W_e)


def _edge_sc_body(x_hbm, src_hbm, dst_hbm, m_hbm, out_hbm,
                  srcA, dstA, srcB, dstB, xA, mA, oA, xB, mB, oB, acc_sh,
                  iA, iB, gmA, gmB, sA, sB):
    c = lax.axis_index("c")
    s = lax.axis_index("s")
    cbase = s * NCHT

    def start_idx(ci, srcr, dstr, q, sem):
        pltpu.async_copy(src_hbm.at[s, ci], srcr.at[q], sem)
        pltpu.async_copy(dst_hbm.at[s, ci], dstr.at[q], sem)

    def wait_idx(ci, srcr, dstr, q, sem):
        pltpu.make_async_copy(src_hbm.at[s, ci], srcr.at[q], sem).wait()
        pltpu.make_async_copy(dst_hbm.at[s, ci], dstr.at[q], sem).wait()

    def start_gm(ci, srcr, q, xbuf, mbuf, sem):
        pltpu.async_copy(x_hbm.at[c].at[srcr.at[q]], xbuf, sem)
        pltpu.async_copy(
            m_hbm.at[c, pl.ds((cbase + ci) * CHT, CHT), :], mbuf, sem)

    def wait_gm(ci, srcr, q, xbuf, mbuf, sem):
        pltpu.make_async_copy(x_hbm.at[c].at[srcr.at[q]], xbuf, sem).wait()
        pltpu.make_async_copy(
            m_hbm.at[c, pl.ds((cbase + ci) * CHT, CHT), :], mbuf, sem).wait()

    def compute(xbuf, mbuf, obuf):
        def row(e, _):
            for j in range(DH // LANES):
                sl = pl.ds(j * LANES, LANES)
                obuf[e, sl] = jnp.maximum(xbuf[e, sl] + mbuf[e, sl], 0.0)
            return 0
        lax.fori_loop(0, CHT, row, 0, unroll=2)

    def start_scatter(dstr, q, obuf, ssem):
        pltpu.async_copy(obuf, acc_sh.at[dstr.at[q]], ssem, add=True)

    def wait_scatter(dstr, q, obuf, ssem):
        pltpu.make_async_copy(obuf, acc_sh.at[dstr.at[q]], ssem).wait()

    start_idx(0, srcA, dstA, 0, iA)
    start_idx(1, srcB, dstB, 0, iB)

    def zrow(i, _):
        for j in range(DH // LANES):
            oA[i, pl.ds(j * LANES, LANES)] = jnp.zeros((LANES,), jnp.float32)
        return 0
    lax.fori_loop(0, RCH, zrow, 0)

    def zcp(i, _):
        pltpu.sync_copy(oA, acc_sh.at[pl.ds(s * RPT + i * RCH, RCH), :])
        return 0
    lax.fori_loop(0, RPT // RCH, zcp, 0)

    wait_idx(0, srcA, dstA, 0, iA)
    start_gm(0, srcA, 0, xA, mA, gmA)
    wait_idx(1, srcB, dstB, 0, iB)
    start_gm(1, srcB, 0, xB, mB, gmB)
    start_idx(2, srcA, dstA, 1, iA)
    start_idx(3, srcB, dstB, 1, iB)
    plsc.subcore_barrier()

    def halfstep(pi, ci, srcr, dstr, xb, mb, ob, isem, gsem, ssem, first):
        q = pi % 3
        if not first:
            wait_scatter(dstr, (pi - 1) % 3, ob, ssem)
        wait_gm(ci, srcr, q, xb, mb, gsem)
        compute(xb, mb, ob)
        start_scatter(dstr, q, ob, ssem)

        @pl.when(ci + 2 < NCHT)
        def _():
            wait_idx(ci + 2, srcr, dstr, (pi + 1) % 3, isem)
            start_gm(ci + 2, srcr, (pi + 1) % 3, xb, mb, gsem)

        @pl.when(ci + 4 < NCHT)
        def _():
            start_idx(ci + 4, srcr, dstr, (pi + 2) % 3, isem)

    halfstep(0, 0, srcA, dstA, xA, mA, oA, iA, gmA, sA, True)
    halfstep(0, 1, srcB, dstB, xB, mB, oB, iB, gmB, sB, True)

    def pair(pi, _):
        halfstep(pi, 2 * pi, srcA, dstA, xA, mA, oA, iA, gmA, sA, False)
        halfstep(pi, 2 * pi + 1, srcB, dstB, xB, mB, oB, iB, gmB, sB, False)
        return 0
    lax.fori_loop(1, NCHT // 2, pair, 0)

    wait_scatter(dstA, (NCHT // 2 - 1) % 3, oA, sA)
    wait_scatter(dstB, (NCHT // 2 - 1) % 3, oB, sB)
    plsc.subcore_barrier()

    def wout(i, _):
        r0 = s * RPT + i * RCH
        pltpu.sync_copy(acc_sh.at[pl.ds(r0, RCH), :], oA)
        pltpu.sync_copy(oA, out_hbm.at[c, pl.ds(r0, RCH), :])
        return 0
    lax.fori_loop(0, RPT // RCH, wout, 0)


def _edge_sc(xs, src3d, dst3d, M):
    mesh = plsc.VectorSubcoreMesh(core_axis_name="c", subcore_axis_name="s",
                                  num_cores=NC, num_subcores=NS)
    return pl.kernel(
        _edge_sc_body,
        out_type=jax.ShapeDtypeStruct((2, NP, DH), jnp.float32),
        mesh=mesh,
        compiler_params=pltpu.CompilerParams(use_tc_tiling_on_sc=False),
        scratch_types=[
            pltpu.VMEM((3, CHT), jnp.int32),
            pltpu.VMEM((3, CHT), jnp.int32),
            pltpu.VMEM((3, CHT), jnp.int32),
            pltpu.VMEM((3, CHT), jnp.int32),
            pltpu.VMEM((CHT, DH), jnp.float32),
            pltpu.VMEM((CHT, DH), jnp.float32),
            pltpu.VMEM((CHT, DH), jnp.float32),
            pltpu.VMEM((CHT, DH), jnp.float32),
            pltpu.VMEM((CHT, DH), jnp.float32),
            pltpu.VMEM((CHT, DH), jnp.float32),
            pltpu.VMEM_SHARED((NP, DH), jnp.float32),
            pltpu.SemaphoreType.DMA,
            pltpu.SemaphoreType.DMA,
            pltpu.SemaphoreType.DMA,
            pltpu.SemaphoreType.DMA,
            pltpu.SemaphoreType.DMA,
            pltpu.SemaphoreType.DMA,
        ],
    )(xs, src3d, dst3d, M)


_R = 2000


def _finish_body(agg_ref, x_ref, b_ref, w1_ref, b1_ref, wh_ref, bh_ref,
                 out_ref, sums_ref, counts_ref):
    i = pl.program_id(0)

    @pl.when(i == 0)
    def _():
        sums_ref[...] = jnp.zeros_like(sums_ref)
        counts_ref[...] = jnp.zeros_like(counts_ref)

    z = jnp.concatenate([agg_ref[0], agg_ref[1]], axis=1) + x_ref[...]
    h = jnp.maximum(
        jnp.dot(z, w1_ref[...], preferred_element_type=jnp.float32)
        + b1_ref[...], 0.0)
    bids = b_ref[0, 0, :]
    gi = lax.broadcasted_iota(jnp.int32, (G, _R), 0)
    oh = (gi == bids[None, :]).astype(jnp.float32)
    sums_ref[...] += jnp.dot(oh, h, preferred_element_type=jnp.float32)
    counts_ref[...] += jnp.sum(oh, axis=1)[None, :]

    @pl.when(i == pl.num_programs(0) - 1)
    def _():
        pooled = sums_ref[...] / jnp.maximum(counts_ref[0, :], 1.0)[:, None]
        out_ref[...] = (jnp.dot(pooled, wh_ref[...],
                                preferred_element_type=jnp.float32)
                        + bh_ref[...])


def _finish(agg2, x, batch3d, W1, b1, W_head, b_head):
    nblk = N // _R
    return pl.pallas_call(
        _finish_body,
        grid=(nblk,),
        in_specs=[
            pl.BlockSpec((2, _R, DH), lambda i: (0, i, 0)),
            pl.BlockSpec((_R, D), lambda i: (i, 0)),
            pl.BlockSpec((1, 1, _R), lambda i: (i, 0, 0)),
            pl.BlockSpec((D, D), lambda i: (0, 0)),
            pl.BlockSpec((1, D), lambda i: (0, 0)),
            pl.BlockSpec((D, T), lambda i: (0, 0)),
            pl.BlockSpec((1, T), lambda i: (0, 0)),
        ],
        out_specs=pl.BlockSpec((G, T), lambda i: (0, 0)),
        out_shape=jax.ShapeDtypeStruct((G, T), jnp.float32),
        scratch_shapes=[
            pltpu.VMEM((G, D), jnp.float32),
            pltpu.VMEM((1, G), jnp.float32),
        ],
    )(agg2, x, batch3d, W1, b1, W_head, b_head)


def kernel(x, edge_index, edge_attr, batch_assignments, W_e, W1, b1,
           W_head, b_head):
    pad = EPAD - E
    src3d = jnp.concatenate(
        [edge_index[0], jnp.zeros((pad,), jnp.int32)]).reshape(NS, NCHT, CHT)
    dst3d = jnp.concatenate(
        [edge_index[1],
         jnp.full((pad,), NP - 1, jnp.int32)]).reshape(NS, NCHT, CHT)
    attr_pad = jnp.concatenate(
        [edge_attr, jnp.zeros((pad, DE), jnp.float32)])
    xs = jnp.stack([x[:, :DH], x[:, DH:]])
    M = _encode(attr_pad, W_e)
    agg2 = _edge_sc(xs, src3d, dst3d, M)
    batch3d = batch_assignments.reshape(N // _R, 1, _R)
    out = _finish(agg2, x, batch3d, W1, b1.reshape(1, D),
                  W_head, b_head.reshape(1, T))
    return out


# speedup vs baseline: 1.4508x; 1.4508x over previous
import functools

import jax
import jax.numpy as jnp
from jax import lax
from jax.experimental import pallas as pl
from jax.experimental.pallas import tpu as pltpu
from jax.experimental.pallas import tpu_sc as plsc

N = 10000
E = 320000
D = 128
DE = 16
G = 512
T = 10

NC = 2
NS = 16
NW = NC * NS
CH = 32
NCHT = 313
EPW = NCHT * CH
EPAD = NW * EPW
NP = 10240
RPT = NP // NS
LANES = 16


_EB = EPAD // 32


def _encode_body(attr_ref, we_ref, out_ref):
    out_ref[...] = jnp.dot(attr_ref[...], we_ref[...],
                           preferred_element_type=jnp.float32)


def _encode(edge_attr, W_e):
    return pl.pallas_call(
        _encode_body,
        grid=(EPAD // _EB,),
        in_specs=[
            pl.BlockSpec((_EB, DE), lambda i: (i, 0)),
            pl.BlockSpec((DE, D), lambda i: (0, 0)),
        ],
        out_specs=pl.BlockSpec((_EB, D), lambda i: (i, 0)),
        out_shape=jax.ShapeDtypeStruct((EPAD, D), jnp.float32),
    )(edge_attr, W_e)


def _edge_sc_body(x_hbm, src_hbm, dst_hbm, m_hbm, out_hbm,
                  srcA, dstA, srcB, dstB, xA, mA, oA, xB, mB, oB, acc_sh,
                  iA, iB, gmA, gmB, sA, sB):
    c = lax.axis_index("c")
    s = lax.axis_index("s")
    wid = s * NC + c
    ebase = wid * EPW

    def start_idx(ci, srcr, dstr, q, sem):
        pltpu.async_copy(src_hbm.at[pl.ds(ebase + ci * CH, CH)],
                         srcr.at[q], sem)
        pltpu.async_copy(dst_hbm.at[pl.ds(ebase + ci * CH, CH)],
                         dstr.at[q], sem)

    def wait_idx(ci, srcr, dstr, q, sem):
        pltpu.make_async_copy(src_hbm.at[pl.ds(ebase + ci * CH, CH)],
                              srcr.at[q], sem).wait()
        pltpu.make_async_copy(dst_hbm.at[pl.ds(ebase + ci * CH, CH)],
                              dstr.at[q], sem).wait()

    def start_gm(ci, srcr, q, xbuf, mbuf, sem):
        pltpu.async_copy(x_hbm.at[srcr.at[q]], xbuf, sem)
        pltpu.async_copy(m_hbm.at[pl.ds(ebase + ci * CH, CH), :], mbuf, sem)

    def wait_gm(ci, srcr, q, xbuf, mbuf, sem):
        pltpu.make_async_copy(x_hbm.at[srcr.at[q]], xbuf, sem).wait()
        pltpu.make_async_copy(m_hbm.at[pl.ds(ebase + ci * CH, CH), :],
                              mbuf, sem).wait()

    def compute(xbuf, mbuf, obuf):
        def row(e, _):
            for j in range(D // LANES):
                sl = pl.ds(j * LANES, LANES)
                obuf[e, sl] = jnp.maximum(xbuf[e, sl] + mbuf[e, sl], 0.0)
            return 0
        lax.fori_loop(0, CH, row, 0, unroll=2)

    def start_scatter(dstr, q, obuf, ssem):
        pltpu.async_copy(obuf, acc_sh.at[dstr.at[q]], ssem, add=True)

    def wait_scatter(dstr, q, obuf, ssem):
        pltpu.make_async_copy(obuf, acc_sh.at[dstr.at[q]], ssem).wait()

    start_idx(0, srcA, dstA, 0, iA)
    start_idx(1, srcB, dstB, 0, iB)

    def zrow(i, _):
        for j in range(D // LANES):
            oA[i, pl.ds(j * LANES, LANES)] = jnp.zeros((LANES,), jnp.float32)
        return 0
    lax.fori_loop(0, CH, zrow, 0)

    def zcp(i, _):
        pltpu.sync_copy(oA, acc_sh.at[pl.ds(s * RPT + i * CH, CH), :])
        return 0
    lax.fori_loop(0, RPT // CH, zcp, 0)

    wait_idx(0, srcA, dstA, 0, iA)
    start_gm(0, srcA, 0, xA, mA, gmA)
    wait_idx(1, srcB, dstB, 0, iB)
    start_gm(1, srcB, 0, xB, mB, gmB)
    start_idx(2, srcA, dstA, 1, iA)
    start_idx(3, srcB, dstB, 1, iB)
    plsc.subcore_barrier()

    def halfstep(pi, ci, srcr, dstr, xb, mb, ob, isem, gsem, ssem, first):
        q = pi % 3
        if not first:
            wait_scatter(dstr, (pi - 1) % 3, ob, ssem)
        wait_gm(ci, srcr, q, xb, mb, gsem)
        compute(xb, mb, ob)
        start_scatter(dstr, q, ob, ssem)

        @pl.when(ci + 2 < NCHT)
        def _():
            wait_idx(ci + 2, srcr, dstr, (pi + 1) % 3, isem)
            start_gm(ci + 2, srcr, (pi + 1) % 3, xb, mb, gsem)

        @pl.when(ci + 4 < NCHT)
        def _():
            start_idx(ci + 4, srcr, dstr, (pi + 2) % 3, isem)

    halfstep(0, 0, srcA, dstA, xA, mA, oA, iA, gmA, sA, True)
    halfstep(0, 1, srcB, dstB, xB, mB, oB, iB, gmB, sB, True)

    def pair(pi, _):
        halfstep(pi, 2 * pi, srcA, dstA, xA, mA, oA, iA, gmA, sA, False)
        halfstep(pi, 2 * pi + 1, srcB, dstB, xB, mB, oB, iB, gmB, sB, False)
        return 0
    lax.fori_loop(1, NCHT // 2, pair, 0)

    pe = NCHT // 2
    halfstep(pe, 2 * pe, srcA, dstA, xA, mA, oA, iA, gmA, sA, False)
    wait_scatter(dstA, pe % 3, oA, sA)
    wait_scatter(dstB, (pe - 1) % 3, oB, sB)
    plsc.subcore_barrier()

    def wout(i, _):
        r0 = s * RPT + i * CH
        pltpu.sync_copy(acc_sh.at[pl.ds(r0, CH), :], oA)
        pltpu.sync_copy(oA, out_hbm.at[c, pl.ds(r0, CH), :])
        return 0
    lax.fori_loop(0, RPT // CH, wout, 0)


def _edge_sc(x, src, dst, M):
    mesh = plsc.VectorSubcoreMesh(core_axis_name="c", subcore_axis_name="s",
                                  num_cores=NC, num_subcores=NS)
    return pl.kernel(
        _edge_sc_body,
        out_type=jax.ShapeDtypeStruct((NC, NP, D), jnp.float32),
        mesh=mesh,
        scratch_types=[
            pltpu.VMEM((3, CH), jnp.int32),
            pltpu.VMEM((3, CH), jnp.int32),
            pltpu.VMEM((3, CH), jnp.int32),
            pltpu.VMEM((3, CH), jnp.int32),
            pltpu.VMEM((CH, D), jnp.float32),
            pltpu.VMEM((CH, D), jnp.float32),
            pltpu.VMEM((CH, D), jnp.float32),
            pltpu.VMEM((CH, D), jnp.float32),
            pltpu.VMEM((CH, D), jnp.float32),
            pltpu.VMEM((CH, D), jnp.float32),
            pltpu.VMEM_SHARED((NP, D), jnp.float32),
            pltpu.SemaphoreType.DMA,
            pltpu.SemaphoreType.DMA,
            pltpu.SemaphoreType.DMA,
            pltpu.SemaphoreType.DMA,
            pltpu.SemaphoreType.DMA,
            pltpu.SemaphoreType.DMA,
        ],
    )(x, src, dst, M)


_R = 2000


def _finish_body(agg_ref, x_ref, b_ref, w1_ref, b1_ref, wh_ref, bh_ref,
                 out_ref, sums_ref, counts_ref):
    i = pl.program_id(0)

    @pl.when(i == 0)
    def _():
        sums_ref[...] = jnp.zeros_like(sums_ref)
        counts_ref[...] = jnp.zeros_like(counts_ref)

    z = agg_ref[0] + agg_ref[1] + x_ref[...]
    h = jnp.maximum(
        jnp.dot(z, w1_ref[...], preferred_element_type=jnp.float32)
        + b1_ref[...], 0.0)
    bids = b_ref[0, 0, :]
    gi = lax.broadcasted_iota(jnp.int32, (G, _R), 0)
    oh = (gi == bids[None, :]).astype(jnp.float32)
    sums_ref[...] += jnp.dot(oh, h, preferred_element_type=jnp.float32)
    counts_ref[...] += jnp.sum(oh, axis=1)[None, :]

    @pl.when(i == pl.num_programs(0) - 1)
    def _():
        pooled = sums_ref[...] / jnp.maximum(counts_ref[0, :], 1.0)[:, None]
        out_ref[...] = (jnp.dot(pooled, wh_ref[...],
                                preferred_element_type=jnp.float32)
                        + bh_ref[...])


def _finish(agg2, x, batch3d, W1, b1, W_head, b_head):
    nblk = N // _R
    return pl.pallas_call(
        _finish_body,
        grid=(nblk,),
        in_specs=[
            pl.BlockSpec((NC, _R, D), lambda i: (0, i, 0)),
            pl.BlockSpec((_R, D), lambda i: (i, 0)),
            pl.BlockSpec((1, 1, _R), lambda i: (i, 0, 0)),
            pl.BlockSpec((D, D), lambda i: (0, 0)),
            pl.BlockSpec((1, D), lambda i: (0, 0)),
            pl.BlockSpec((D, T), lambda i: (0, 0)),
            pl.BlockSpec((1, T), lambda i: (0, 0)),
        ],
        out_specs=pl.BlockSpec((G, T), lambda i: (0, 0)),
        out_shape=jax.ShapeDtypeStruct((G, T), jnp.float32),
        scratch_shapes=[
            pltpu.VMEM((G, D), jnp.float32),
            pltpu.VMEM((1, G), jnp.float32),
        ],
    )(agg2, x, batch3d, W1, b1, W_head, b_head)


def kernel(x, edge_index, edge_attr, batch_assignments, W_e, W1, b1,
           W_head, b_head):
    pad = EPAD - E
    src = jnp.concatenate([edge_index[0], jnp.zeros((pad,), jnp.int32)])
    dst = jnp.concatenate([edge_index[1],
                           jnp.full((pad,), NP - 1, jnp.int32)])
    attr_pad = jnp.concatenate([edge_attr, jnp.zeros((pad, DE), jnp.float32)])
    M = _encode(attr_pad, W_e)
    agg2 = _edge_sc(x, src, dst, M)
    batch3d = batch_assignments.reshape(N // _R, 1, _R)
    out = _finish(agg2, x, batch3d, W1, b1.reshape(1, D),
                  W_head, b_head.reshape(1, T))
    return out
